# baseline (speedup 1.0000x reference)
import functools

import jax
import jax.numpy as jnp
from jax import lax
from jax.experimental import pallas as pl
from jax.experimental.pallas import tpu as pltpu
from jax.experimental.pallas import tpu_sc as plsc


def _sc_cnt_body(con_v, idx_v, one_v, zero_v, acc_sp, out_hbm, n, e):
    for i in range(e // 16):
        s = con_v[0, pl.ds(i * 16, 16)]
        d = con_v[1, pl.ds(i * 16, 16)]
        idx_v[pl.ds(i * 16, 16)] = d * n + s
        one_v[pl.ds(i * 16, 16)] = jnp.full((16,), 1.0, jnp.float32)
    for i in range(n * n // 16):
        zero_v[pl.ds(i * 16, 16)] = jnp.zeros((16,), jnp.float32)
    pltpu.sync_copy(zero_v, acc_sp)
    pltpu.sync_copy(one_v, acc_sp.at[idx_v], add=True)
    pltpu.sync_copy(acc_sp, zero_v)
    pltpu.sync_copy(zero_v, out_hbm)


def _sc_cnt_kernel(cha_hbm, spa_hbm, c1_hbm, c2_hbm,
                   cha_v, idx1_v, one1_v, z1_v,
                   spa_v, idx2_v, one2_v, z2_v, acc1_sp, acc2_sp):
    @pl.when((lax.axis_index("c") == 0) & (lax.axis_index("s") == 0))
    def _():
        pltpu.sync_copy(cha_hbm, cha_v)
        pltpu.sync_copy(spa_hbm, spa_v)
        _sc_cnt_body(cha_v, idx1_v, one1_v, z1_v, acc1_sp, c1_hbm, 64, 2048)
        _sc_cnt_body(spa_v, idx2_v, one2_v, z2_v, acc2_sp, c2_hbm, 36, 1024)


def _sc_cnt(cha_con, spa_con):
    mesh = plsc.VectorSubcoreMesh(core_axis_name="c", subcore_axis_name="s")
    run = pl.kernel(
        _sc_cnt_kernel, mesh=mesh,
        out_type=[jax.ShapeDtypeStruct((4096,), jnp.float32),
                  jax.ShapeDtypeStruct((1296,), jnp.float32)],
        scratch_types=[
            pltpu.VMEM((2, 2048), jnp.int32),
            pltpu.VMEM((2048,), jnp.int32),
            pltpu.VMEM((2048,), jnp.float32),
            pltpu.VMEM((4096,), jnp.float32),
            pltpu.VMEM((2, 1024), jnp.int32),
            pltpu.VMEM((1024,), jnp.int32),
            pltpu.VMEM((1024,), jnp.float32),
            pltpu.VMEM((1296,), jnp.float32),
            pltpu.VMEM_SHARED((4096,), jnp.float32),
            pltpu.VMEM_SHARED((1296,), jnp.float32),
        ])
    c1, c2 = run(cha_con, spa_con)
    return c1.reshape(64, 64), c2.reshape(36, 36)

_NEG = -1e30


def _edge_cnt(con, n):
    e = con.shape[1]
    iota = jax.lax.broadcasted_iota(jnp.int32, (n, e), 0)
    a = (con[1:2, :] == iota).astype(jnp.float32)
    b = (con[0:1, :] == iota).astype(jnp.float32)
    return jax.lax.dot_general(
        a, b, (((1,), (1,)), ((), ())), preferred_element_type=jnp.float32)


def _gat_dense(x, W, Al, Ar, SD, cnt, H, D):
    nb, n, f = x.shape
    h = jax.lax.dot_general(
        x, W, (((2,), (1,)), ((), ())), preferred_element_type=jnp.float32)
    el = jax.lax.dot_general(
        h, Al, (((2,), (0,)), ((), ())), preferred_element_type=jnp.float32)
    er = jax.lax.dot_general(
        h, Ar, (((2,), (0,)), ((), ())), preferred_element_type=jnp.float32)
    elt = el.transpose(0, 2, 1)
    ert = er.transpose(0, 2, 1)
    e = elt[:, :, None, :] + ert[:, :, :, None]
    e = jnp.maximum(e, 0.2 * e)
    ex = jnp.exp(e) * cnt[None, None, :, :]
    denom = jnp.sum(ex, axis=-1)
    rden = jnp.where(denom > 0, 1.0 / denom, 0.0)
    parts = []
    for hd in range(H):
        parts.append(rden[:, hd, :, None] * jax.lax.dot_general(
            ex[:, hd], h[:, :, hd * D:(hd + 1) * D],
            (((2,), (1,)), ((0,), (0,))),
            preferred_element_type=jnp.float32))
    return jnp.concatenate(parts, axis=-1)


def _main_kernel(nd_ref, c1_ref, c2_ref, wc_ref, alc_ref, arc_ref, bc_ref,
                 sdc_ref, ws_ref, als_ref, ars_ref, bs_ref, sds_ref, out_ref):
    cnt1 = c1_ref[...]
    cnt2 = c2_ref[...]
    x = nd_ref[...]
    r1 = _gat_dense(x, wc_ref[...], alc_ref[...], arc_ref[...],
                    sdc_ref[...], cnt1, 6, 6)
    r1 = r1 + bc_ref[...]
    y = r1.transpose(0, 2, 1)
    r2 = _gat_dense(y, ws_ref[...], als_ref[...], ars_ref[...],
                    sds_ref[...], cnt2, 8, 8)
    r2 = r2 + bs_ref[...]
    out_ref[...] = r2.transpose(0, 2, 1)


def _whole(shape):
    nd = len(shape)
    return pl.BlockSpec(shape, lambda i: (0,) * nd)


@functools.partial(jax.jit, static_argnames=("interpret",))
def _run(ndata, cha_con, spa_con, W_cha, al_cha, ar_cha, b_cha,
         W_spa, al_spa, ar_spa, b_spa, interpret=False):
    nd = ndata.reshape(512, 64, 36)
    cnt1, cnt2 = _sc_cnt(cha_con.reshape(2, 2048), spa_con.reshape(2, 1024))
    eye6 = jnp.eye(6, dtype=jnp.float32)
    alc = (al_cha[0][:, :, None] * eye6[:, None, :]).reshape(36, 6)
    arc = (ar_cha[0][:, :, None] * eye6[:, None, :]).reshape(36, 6)
    eye8 = jnp.eye(8, dtype=jnp.float32)
    als = (al_spa[0][:, :, None] * eye8[:, None, :]).reshape(64, 8)
    ars = (ar_spa[0][:, :, None] * eye8[:, None, :]).reshape(64, 8)
    sdc = jnp.repeat(eye6, 6, axis=1)
    sds = jnp.repeat(eye8, 8, axis=1)

    NB = 64
    out = pl.pallas_call(
        _main_kernel,
        grid=(512 // NB,),
        in_specs=[
            pl.BlockSpec((NB, 64, 36), lambda i: (i, 0, 0)),
            _whole((64, 64)), _whole((36, 36)),
            _whole((36, 36)), _whole((36, 6)), _whole((36, 6)), _whole((1, 36)),
            _whole((6, 36)),
            _whole((64, 64)), _whole((64, 8)), _whole((64, 8)), _whole((1, 64)),
            _whole((8, 64)),
        ],
        out_specs=pl.BlockSpec((NB, 64, 36), lambda i: (i, 0, 0)),
        out_shape=jax.ShapeDtypeStruct((512, 64, 36), jnp.float32),
        interpret=interpret,
    )(nd, cnt1, cnt2,
      W_cha, alc, arc, b_cha.reshape(1, 36), sdc,
      W_spa, als, ars, b_spa.reshape(1, 64), sds)
    return out.reshape(4, 8, 16, 64, 36)


def kernel(ndata, cha_con, spa_con, W_cha, al_cha, ar_cha, b_cha,
           W_spa, al_spa, ar_spa, b_spa):
    return _run(ndata, cha_con, spa_con, W_cha, al_cha, ar_cha, b_cha,
                W_spa, al_spa, ar_spa, b_spa)


# speedup vs baseline: 1.0012x; 1.0012x over previous
import functools

import jax
import jax.numpy as jnp
from jax import lax
from jax.experimental import pallas as pl
from jax.experimental.pallas import tpu as pltpu
from jax.experimental.pallas import tpu_sc as plsc


def _sc_cnt_kernel(cha_hbm, spa_hbm, c1_hbm, c2_hbm,
                   cha_v, idx1_v, one1_v, z1_v,
                   spa_v, idx2_v, one2_v, z2_v, acc1_sp, acc2_sp):
    sid = lax.axis_index("s")

    @pl.when(lax.axis_index("c") == 0)
    def _():
        pltpu.sync_copy(cha_hbm, cha_v)
        pltpu.sync_copy(spa_hbm, spa_v)
        for i in range(8):
            off = sid * 128 + i * 16
            s = cha_v[0, pl.ds(off, 16)]
            d = cha_v[1, pl.ds(off, 16)]
            idx1_v[pl.ds(i * 16, 16)] = d * 64 + s
            one1_v[pl.ds(i * 16, 16)] = jnp.full((16,), 1.0, jnp.float32)
        for i in range(4):
            off = sid * 64 + i * 16
            s = spa_v[0, pl.ds(off, 16)]
            d = spa_v[1, pl.ds(off, 16)]
            idx2_v[pl.ds(i * 16, 16)] = d * 36 + s
            one2_v[pl.ds(i * 16, 16)] = jnp.full((16,), 1.0, jnp.float32)

        @pl.when(sid == 0)
        def _():
            for i in range(256):
                z1_v[pl.ds(i * 16, 16)] = jnp.zeros((16,), jnp.float32)
            for i in range(81):
                z2_v[pl.ds(i * 16, 16)] = jnp.zeros((16,), jnp.float32)
            pltpu.sync_copy(z1_v, acc1_sp)
            pltpu.sync_copy(z2_v, acc2_sp)

        plsc.subcore_barrier()
        pltpu.sync_copy(one1_v, acc1_sp.at[idx1_v], add=True)
        pltpu.sync_copy(one2_v, acc2_sp.at[idx2_v], add=True)
        plsc.subcore_barrier()

        @pl.when(sid == 0)
        def _():
            pltpu.sync_copy(acc1_sp, z1_v)
            pltpu.sync_copy(z1_v, c1_hbm)
            pltpu.sync_copy(acc2_sp, z2_v)
            pltpu.sync_copy(z2_v, c2_hbm)


def _sc_cnt(cha_con, spa_con):
    mesh = plsc.VectorSubcoreMesh(core_axis_name="c", subcore_axis_name="s")
    run = pl.kernel(
        _sc_cnt_kernel, mesh=mesh,
        out_type=[jax.ShapeDtypeStruct((4096,), jnp.float32),
                  jax.ShapeDtypeStruct((1296,), jnp.float32)],
        scratch_types=[
            pltpu.VMEM((2, 2048), jnp.int32),
            pltpu.VMEM((128,), jnp.int32),
            pltpu.VMEM((128,), jnp.float32),
            pltpu.VMEM((4096,), jnp.float32),
            pltpu.VMEM((2, 1024), jnp.int32),
            pltpu.VMEM((64,), jnp.int32),
            pltpu.VMEM((64,), jnp.float32),
            pltpu.VMEM((1296,), jnp.float32),
            pltpu.VMEM_SHARED((4096,), jnp.float32),
            pltpu.VMEM_SHARED((1296,), jnp.float32),
        ])
    c1, c2 = run(cha_con, spa_con)
    return c1.reshape(64, 64), c2.reshape(36, 36)

_NEG = -1e30


def _edge_cnt(con, n):
    e = con.shape[1]
    iota = jax.lax.broadcasted_iota(jnp.int32, (n, e), 0)
    a = (con[1:2, :] == iota).astype(jnp.float32)
    b = (con[0:1, :] == iota).astype(jnp.float32)
    return jax.lax.dot_general(
        a, b, (((1,), (1,)), ((), ())), preferred_element_type=jnp.float32)


def _gat_dense(x, W, Al, Ar, SD, cnt, H, D):
    nb, n, f = x.shape
    h = jax.lax.dot_general(
        x, W, (((2,), (1,)), ((), ())), preferred_element_type=jnp.float32)
    el = jax.lax.dot_general(
        h, Al, (((2,), (0,)), ((), ())), preferred_element_type=jnp.float32)
    er = jax.lax.dot_general(
        h, Ar, (((2,), (0,)), ((), ())), preferred_element_type=jnp.float32)
    elt = el.transpose(0, 2, 1)
    ert = er.transpose(0, 2, 1)
    e = elt[:, :, None, :] + ert[:, :, :, None]
    e = jnp.maximum(e, 0.2 * e)
    ex = jnp.exp(e) * cnt[None, None, :, :]
    denom = jnp.sum(ex, axis=-1)
    rden = jnp.where(denom > 0, 1.0 / denom, 0.0)
    parts = []
    for hd in range(H):
        parts.append(rden[:, hd, :, None] * jax.lax.dot_general(
            ex[:, hd], h[:, :, hd * D:(hd + 1) * D],
            (((2,), (1,)), ((0,), (0,))),
            preferred_element_type=jnp.float32))
    return jnp.concatenate(parts, axis=-1)


def _main_kernel(nd_ref, c1_ref, c2_ref, wc_ref, alc_ref, arc_ref, bc_ref,
                 sdc_ref, ws_ref, als_ref, ars_ref, bs_ref, sds_ref, out_ref):
    cnt1 = c1_ref[...]
    cnt2 = c2_ref[...]
    x = nd_ref[...]
    r1 = _gat_dense(x, wc_ref[...], alc_ref[...], arc_ref[...],
                    sdc_ref[...], cnt1, 6, 6)
    r1 = r1 + bc_ref[...]
    y = r1.transpose(0, 2, 1)
    r2 = _gat_dense(y, ws_ref[...], als_ref[...], ars_ref[...],
                    sds_ref[...], cnt2, 8, 8)
    r2 = r2 + bs_ref[...]
    out_ref[...] = r2.transpose(0, 2, 1)


def _whole(shape):
    nd = len(shape)
    return pl.BlockSpec(shape, lambda i: (0,) * nd)


@functools.partial(jax.jit, static_argnames=("interpret",))
def _run(ndata, cha_con, spa_con, W_cha, al_cha, ar_cha, b_cha,
         W_spa, al_spa, ar_spa, b_spa, interpret=False):
    nd = ndata.reshape(512, 64, 36)
    cnt1, cnt2 = _sc_cnt(cha_con.reshape(2, 2048), spa_con.reshape(2, 1024))
    eye6 = jnp.eye(6, dtype=jnp.float32)
    alc = (al_cha[0][:, :, None] * eye6[:, None, :]).reshape(36, 6)
    arc = (ar_cha[0][:, :, None] * eye6[:, None, :]).reshape(36, 6)
    eye8 = jnp.eye(8, dtype=jnp.float32)
    als = (al_spa[0][:, :, None] * eye8[:, None, :]).reshape(64, 8)
    ars = (ar_spa[0][:, :, None] * eye8[:, None, :]).reshape(64, 8)
    sdc = jnp.repeat(eye6, 6, axis=1)
    sds = jnp.repeat(eye8, 8, axis=1)

    NB = 64
    out = pl.pallas_call(
        _main_kernel,
        grid=(512 // NB,),
        in_specs=[
            pl.BlockSpec((NB, 64, 36), lambda i: (i, 0, 0)),
            _whole((64, 64)), _whole((36, 36)),
            _whole((36, 36)), _whole((36, 6)), _whole((36, 6)), _whole((1, 36)),
            _whole((6, 36)),
            _whole((64, 64)), _whole((64, 8)), _whole((64, 8)), _whole((1, 64)),
            _whole((8, 64)),
        ],
        out_specs=pl.BlockSpec((NB, 64, 36), lambda i: (i, 0, 0)),
        out_shape=jax.ShapeDtypeStruct((512, 64, 36), jnp.float32),
        interpret=interpret,
    )(nd, cnt1, cnt2,
      W_cha, alc, arc, b_cha.reshape(1, 36), sdc,
      W_spa, als, ars, b_spa.reshape(1, 64), sds)
    return out.reshape(4, 8, 16, 64, 36)


def kernel(ndata, cha_con, spa_con, W_cha, al_cha, ar_cha, b_cha,
           W_spa, al_spa, ar_spa, b_spa):
    return _run(ndata, cha_con, spa_con, W_cha, al_cha, ar_cha, b_cha,
                W_spa, al_spa, ar_spa, b_spa)


# speedup vs baseline: 1.0013x; 1.0001x over previous
import functools

import jax
import jax.numpy as jnp
from jax import lax
from jax.experimental import pallas as pl
from jax.experimental.pallas import tpu as pltpu
from jax.experimental.pallas import tpu_sc as plsc


def _sc_cnt_kernel(cha_hbm, spa_hbm, c1_hbm, c2_hbm,
                   cha_v, idx1_v, one1_v, z1_v,
                   spa_v, idx2_v, one2_v, z2_v, acc1_sp, acc2_sp):
    sid = lax.axis_index("s")

    @pl.when(lax.axis_index("c") == 0)
    def _():
        pltpu.sync_copy(cha_hbm, cha_v)
        pltpu.sync_copy(spa_hbm, spa_v)
        for i in range(8):
            off = sid * 128 + i * 16
            s = cha_v[0, pl.ds(off, 16)]
            d = cha_v[1, pl.ds(off, 16)]
            idx1_v[pl.ds(i * 16, 16)] = d * 64 + s
            one1_v[pl.ds(i * 16, 16)] = jnp.full((16,), 1.0, jnp.float32)
        for i in range(4):
            off = sid * 64 + i * 16
            s = spa_v[0, pl.ds(off, 16)]
            d = spa_v[1, pl.ds(off, 16)]
            idx2_v[pl.ds(i * 16, 16)] = d * 36 + s
            one2_v[pl.ds(i * 16, 16)] = jnp.full((16,), 1.0, jnp.float32)

        @pl.when(sid == 0)
        def _():
            for i in range(256):
                z1_v[pl.ds(i * 16, 16)] = jnp.zeros((16,), jnp.float32)
            for i in range(81):
                z2_v[pl.ds(i * 16, 16)] = jnp.zeros((16,), jnp.float32)
            pltpu.sync_copy(z1_v, acc1_sp)
            pltpu.sync_copy(z2_v, acc2_sp)

        plsc.subcore_barrier()
        pltpu.sync_copy(one1_v, acc1_sp.at[idx1_v], add=True)
        pltpu.sync_copy(one2_v, acc2_sp.at[idx2_v], add=True)
        plsc.subcore_barrier()

        @pl.when(sid == 0)
        def _():
            pltpu.sync_copy(acc1_sp, z1_v)
            pltpu.sync_copy(z1_v, c1_hbm)
            pltpu.sync_copy(acc2_sp, z2_v)
            pltpu.sync_copy(z2_v, c2_hbm)


def _sc_cnt(cha_con, spa_con):
    mesh = plsc.VectorSubcoreMesh(core_axis_name="c", subcore_axis_name="s")
    run = pl.kernel(
        _sc_cnt_kernel, mesh=mesh,
        out_type=[jax.ShapeDtypeStruct((4096,), jnp.float32),
                  jax.ShapeDtypeStruct((1296,), jnp.float32)],
        scratch_types=[
            pltpu.VMEM((2, 2048), jnp.int32),
            pltpu.VMEM((128,), jnp.int32),
            pltpu.VMEM((128,), jnp.float32),
            pltpu.VMEM((4096,), jnp.float32),
            pltpu.VMEM((2, 1024), jnp.int32),
            pltpu.VMEM((64,), jnp.int32),
            pltpu.VMEM((64,), jnp.float32),
            pltpu.VMEM((1296,), jnp.float32),
            pltpu.VMEM_SHARED((4096,), jnp.float32),
            pltpu.VMEM_SHARED((1296,), jnp.float32),
        ])
    c1, c2 = run(cha_con, spa_con)
    return c1.reshape(64, 64), c2.reshape(36, 36)

def _gat_dense(x, W, Al, Ar, SD, cnt, H, D):
    nb, n, f = x.shape
    h = jax.lax.dot_general(
        x, W, (((2,), (1,)), ((), ())), preferred_element_type=jnp.float32)
    el = jax.lax.dot_general(
        h, Al, (((2,), (0,)), ((), ())), preferred_element_type=jnp.float32)
    er = jax.lax.dot_general(
        h, Ar, (((2,), (0,)), ((), ())), preferred_element_type=jnp.float32)
    elt = el.transpose(0, 2, 1)
    ert = er.transpose(0, 2, 1)
    e = elt[:, :, None, :] + ert[:, :, :, None]
    e = jnp.maximum(e, 0.2 * e)
    ex = jnp.exp(e) * cnt[None, None, :, :]
    denom = jnp.sum(ex, axis=-1)
    rden = jnp.where(denom > 0, 1.0 / denom, 0.0)
    parts = []
    for hd in range(H):
        parts.append(rden[:, hd, :, None] * jax.lax.dot_general(
            ex[:, hd], h[:, :, hd * D:(hd + 1) * D],
            (((2,), (1,)), ((0,), (0,))),
            preferred_element_type=jnp.float32))
    return jnp.concatenate(parts, axis=-1)


def _main_kernel(nd_ref, c1_ref, c2_ref, wc_ref, alc_ref, arc_ref, bc_ref,
                 sdc_ref, ws_ref, als_ref, ars_ref, bs_ref, sds_ref, out_ref):
    cnt1 = c1_ref[...]
    cnt2 = c2_ref[...]
    x = nd_ref[...]
    r1 = _gat_dense(x, wc_ref[...], alc_ref[...], arc_ref[...],
                    sdc_ref[...], cnt1, 6, 6)
    r1 = r1 + bc_ref[...]
    y = r1.transpose(0, 2, 1)
    r2 = _gat_dense(y, ws_ref[...], als_ref[...], ars_ref[...],
                    sds_ref[...], cnt2, 8, 8)
    r2 = r2 + bs_ref[...]
    out_ref[...] = r2.transpose(0, 2, 1)


def _whole(shape):
    nd = len(shape)
    return pl.BlockSpec(shape, lambda i: (0,) * nd)


@jax.jit
def _run(ndata, cha_con, spa_con, W_cha, al_cha, ar_cha, b_cha,
         W_spa, al_spa, ar_spa, b_spa):
    nd = ndata.reshape(512, 64, 36)
    cnt1, cnt2 = _sc_cnt(cha_con.reshape(2, 2048), spa_con.reshape(2, 1024))
    eye6 = jnp.eye(6, dtype=jnp.float32)
    alc = (al_cha[0][:, :, None] * eye6[:, None, :]).reshape(36, 6)
    arc = (ar_cha[0][:, :, None] * eye6[:, None, :]).reshape(36, 6)
    eye8 = jnp.eye(8, dtype=jnp.float32)
    als = (al_spa[0][:, :, None] * eye8[:, None, :]).reshape(64, 8)
    ars = (ar_spa[0][:, :, None] * eye8[:, None, :]).reshape(64, 8)
    sdc = jnp.repeat(eye6, 6, axis=1)
    sds = jnp.repeat(eye8, 8, axis=1)

    NB = 64
    out = pl.pallas_call(
        _main_kernel,
        grid=(512 // NB,),
        in_specs=[
            pl.BlockSpec((NB, 64, 36), lambda i: (i, 0, 0)),
            _whole((64, 64)), _whole((36, 36)),
            _whole((36, 36)), _whole((36, 6)), _whole((36, 6)), _whole((1, 36)),
            _whole((6, 36)),
            _whole((64, 64)), _whole((64, 8)), _whole((64, 8)), _whole((1, 64)),
            _whole((8, 64)),
        ],
        out_specs=pl.BlockSpec((NB, 64, 36), lambda i: (i, 0, 0)),
        out_shape=jax.ShapeDtypeStruct((512, 64, 36), jnp.float32),
    )(nd, cnt1, cnt2,
      W_cha, alc, arc, b_cha.reshape(1, 36), sdc,
      W_spa, als, ars, b_spa.reshape(1, 64), sds)
    return out.reshape(4, 8, 16, 64, 36)


def kernel(ndata, cha_con, spa_con, W_cha, al_cha, ar_cha, b_cha,
           W_spa, al_spa, ar_spa, b_spa):
    return _run(ndata, cha_con, spa_con, W_cha, al_cha, ar_cha, b_cha,
                W_spa, al_spa, ar_spa, b_spa)
